# baseline (speedup 1.0000x reference)
import functools

import jax
import jax.numpy as jnp
from jax import lax
from jax.experimental import pallas as pl
from jax.experimental.pallas import tpu as pltpu
from jax.experimental.pallas import tpu_sc as plsc

_BATCH = 16384
_HIST = 200
_DIM = 64
_N_ROWS = _BATCH * _HIST

_info = plsc.get_sparse_core_info()
_NC, _NS = _info.num_cores, _info.num_subcores
_NW = _NC * _NS
_PER_W = _N_ROWS // _NW
_CHUNK = 512
_STEPS = _PER_W // _CHUNK


@functools.partial(
    pl.kernel,
    mesh=plsc.VectorSubcoreMesh(core_axis_name="c", subcore_axis_name="s"),
    out_type=jax.ShapeDtypeStruct((_N_ROWS, _DIM), jnp.float32),
    scratch_types=[
        pltpu.VMEM((_CHUNK,), jnp.int32),
        pltpu.VMEM((_CHUNK, _DIM), jnp.float32),
        pltpu.SemaphoreType.DMA,
    ],
    compiler_params=pltpu.CompilerParams(use_tc_tiling_on_sc=False),
)
def _sc_gather(idx_hbm, tab_hbm, out_hbm, idx_v, rows_v, sem):
    wid = lax.axis_index("s") * _NC + lax.axis_index("c")
    base = wid * _PER_W

    def step(g, carry):
        off = base + g * _CHUNK
        pltpu.sync_copy(idx_hbm.at[pl.ds(off, _CHUNK)], idx_v)
        pltpu.async_copy(tab_hbm.at[idx_v], rows_v, sem).wait()
        pltpu.sync_copy(rows_v, out_hbm.at[pl.ds(off, _CHUNK)])
        return carry

    lax.fori_loop(0, _STEPS, step, jnp.int32(0))


def kernel(a, embed_weight):
    idx = a.reshape(-1).astype(jnp.int32)
    out = _sc_gather(idx, embed_weight)
    return out.reshape(a.shape + (embed_weight.shape[1],))


# speedup vs baseline: 1.0690x; 1.0690x over previous
import functools

import jax
import jax.numpy as jnp
from jax import lax
from jax.experimental import pallas as pl
from jax.experimental.pallas import tpu as pltpu
from jax.experimental.pallas import tpu_sc as plsc

_BATCH = 16384
_HIST = 200
_DIM = 64
_N_ROWS = _BATCH * _HIST

_info = plsc.get_sparse_core_info()
_NC, _NS = _info.num_cores, _info.num_subcores
_NW = _NC * _NS
_PER_W = _N_ROWS // _NW
_CHUNK = 800
_STEPS = _PER_W // _CHUNK
_NPAIR = _STEPS // 2


@functools.partial(
    pl.kernel,
    mesh=plsc.VectorSubcoreMesh(core_axis_name="c", subcore_axis_name="s"),
    out_type=jax.ShapeDtypeStruct((_N_ROWS, _DIM), jnp.float32),
    scratch_types=[
        pltpu.VMEM((_CHUNK,), jnp.int32),
        pltpu.VMEM((_CHUNK,), jnp.int32),
        pltpu.VMEM((_CHUNK, _DIM), jnp.float32),
        pltpu.VMEM((_CHUNK, _DIM), jnp.float32),
        pltpu.SemaphoreType.DMA,
        pltpu.SemaphoreType.DMA,
        pltpu.SemaphoreType.DMA,
        pltpu.SemaphoreType.DMA,
    ],
    compiler_params=pltpu.CompilerParams(use_tc_tiling_on_sc=False),
)
def _sc_gather(idx_hbm, tab_hbm, out_hbm, idx_v0, idx_v1, rows_v0, rows_v1,
               gsem0, gsem1, wsem0, wsem1):
    wid = lax.axis_index("s") * _NC + lax.axis_index("c")
    base = wid * _PER_W

    def gather(idx_v, rows_v, gsem):
        return pltpu.make_async_copy(tab_hbm.at[idx_v], rows_v, gsem)

    def writeout(rows_v, off, wsem):
        return pltpu.make_async_copy(rows_v, out_hbm.at[pl.ds(off, _CHUNK)], wsem)

    pltpu.sync_copy(idx_hbm.at[pl.ds(base, _CHUNK)], idx_v0)
    gather(idx_v0, rows_v0, gsem0).start()

    def pair(i, carry):
        off0 = base + (2 * i) * _CHUNK
        off1 = off0 + _CHUNK

        gather(idx_v0, rows_v0, gsem0).wait()
        writeout(rows_v0, off0, wsem0).start()

        @pl.when(i > 0)
        def _():
            writeout(rows_v1, off0 - _CHUNK, wsem1).wait()

        pltpu.sync_copy(idx_hbm.at[pl.ds(off1, _CHUNK)], idx_v1)
        gather(idx_v1, rows_v1, gsem1).start()

        gather(idx_v1, rows_v1, gsem1).wait()
        writeout(rows_v1, off1, wsem1).start()

        writeout(rows_v0, off0, wsem0).wait()

        @pl.when(i < _NPAIR - 1)
        def _():
            pltpu.sync_copy(idx_hbm.at[pl.ds(off1 + _CHUNK, _CHUNK)], idx_v0)
            gather(idx_v0, rows_v0, gsem0).start()

        return carry

    lax.fori_loop(0, _NPAIR, pair, jnp.int32(0))

    writeout(rows_v1, base + (_STEPS - 1) * _CHUNK, wsem1).wait()


def kernel(a, embed_weight):
    idx = a.reshape(-1).astype(jnp.int32)
    out = _sc_gather(idx, embed_weight)
    return out.reshape(a.shape + (embed_weight.shape[1],))
